# baseline (speedup 1.0000x reference)
import functools

import jax
import jax.numpy as jnp
import numpy as np
from jax import lax
from jax.experimental import pallas as pl
from jax.experimental.pallas import tpu as pltpu
from jax.experimental.pallas import tpu_sc as plsc

_B, _N, _C, _K = 8, 2048, 128, 16
_RB = 256
_NBLK = _N // _RB
_SP = 128


def _dotT(a, b):
    return lax.dot_general(a, b, (((1,), (1,)), ((), ())),
                           preferred_element_type=jnp.float32)


def _topk16(d2, b, iota):
    m = jnp.min(d2, axis=1, keepdims=True)
    ji = jnp.min(jnp.where(d2 <= m, iota, _N), axis=1, keepdims=True)
    cols = [ji]
    for _ in range(_K - 1):
        live = (d2 > m) | ((d2 == m) & (iota > ji))
        wf = jnp.where(live, d2, jnp.inf)
        m = jnp.min(wf, axis=1, keepdims=True)
        ji = jnp.min(jnp.where(wf <= m, iota, _N), axis=1, keepdims=True)
        cols.append(ji)
    return jnp.concatenate(cols, axis=1) + b * _N


def _knn1_kernel(srcb_ref, srcf_ref, idx_ref):
    b = pl.program_id(0)
    sb = srcb_ref[0]
    sf = srcf_ref[0]
    x2b = jnp.sum(sb * sb, axis=-1, keepdims=True)
    x2f = jnp.sum(sf * sf, axis=-1, keepdims=True)
    d2 = x2b + x2f.reshape(1, _N) - 2.0 * _dotT(sb, sf)
    d2 = jnp.maximum(d2, 0.0)
    iota = lax.broadcasted_iota(jnp.int32, (_RB, _N), 1)
    idx_ref[0] = _topk16(d2, b, iota)


def _knn1(src):
    return pl.pallas_call(
        _knn1_kernel,
        grid=(_B, _NBLK),
        in_specs=[
            pl.BlockSpec((1, _RB, 3), lambda b, i: (b, i, 0)),
            pl.BlockSpec((1, _N, 3), lambda b, i: (b, 0, 0)),
        ],
        out_specs=pl.BlockSpec((1, _RB, _K), lambda b, i: (b, i, 0)),
        out_shape=jax.ShapeDtypeStruct((_B, _N, _K), jnp.int32),
    )(src, src)


def _knn2_kernel(ob_ref, of_ref, wq1_ref, idx_ref, q2_ref):
    b = pl.program_id(0)
    ob = ob_ref[0]
    of = of_ref[0]
    x2b = jnp.sum(ob * ob, axis=-1, keepdims=True)
    x2f = jnp.sum(of * of, axis=-1, keepdims=True)
    d2 = x2b + x2f.reshape(1, _N) - 2.0 * _dotT(ob, of)
    d2 = jnp.maximum(d2, 0.0)
    iota = lax.broadcasted_iota(jnp.int32, (_RB, _N), 1)
    idx_ref[0] = _topk16(d2, b, iota)
    q2_ref[0] = _dotT(ob, wq1_ref[...])


def _knn2(out1, Wq1):
    return pl.pallas_call(
        _knn2_kernel,
        grid=(_B, _NBLK),
        in_specs=[
            pl.BlockSpec((1, _RB, _C), lambda b, i: (b, i, 0)),
            pl.BlockSpec((1, _N, _C), lambda b, i: (b, 0, 0)),
            pl.BlockSpec((_C, _C), lambda b, i: (0, 0)),
        ],
        out_specs=[
            pl.BlockSpec((1, _RB, _K), lambda b, i: (b, i, 0)),
            pl.BlockSpec((1, _RB, _C), lambda b, i: (b, i, 0)),
        ],
        out_shape=[
            jax.ShapeDtypeStruct((_B, _N, _K), jnp.int32),
            jax.ShapeDtypeStruct((_B, _N, _C), jnp.float32),
        ],
    )(out1, out1, Wq1)


_NWORK = 32
_CH = 512


def _sc_gather1(tgt2d, srcp2d, idx):
    tot = idx.shape[0]
    per_w = tot // _NWORK
    n_chunks = per_w // _CH
    mesh = plsc.VectorSubcoreMesh(core_axis_name="c", subcore_axis_name="s")

    @functools.partial(
        pl.kernel, mesh=mesh,
        out_type=[
            jax.ShapeDtypeStruct((tot, _C), jnp.float32),
            jax.ShapeDtypeStruct((tot, _SP), jnp.float32),
        ],
        scratch_types=[
            pltpu.VMEM((_CH,), jnp.int32),
            pltpu.VMEM((_CH, _C), jnp.float32),
            pltpu.SemaphoreType.DMA,
        ],
    )
    def gk(tgt_hbm, srcp_hbm, idx_hbm, ot_hbm, os_hbm,
           idx_v, rows_v, sem):
        wid = lax.axis_index("s") * 2 + lax.axis_index("c")
        base = wid * per_w

        def body(i, carry):
            off = base + i * _CH
            pltpu.sync_copy(idx_hbm.at[pl.ds(off, _CH)], idx_v)
            pltpu.async_copy(tgt_hbm.at[idx_v], rows_v, sem).wait()
            pltpu.sync_copy(rows_v, ot_hbm.at[pl.ds(off, _CH)])
            pltpu.async_copy(srcp_hbm.at[idx_v], rows_v, sem).wait()
            pltpu.sync_copy(rows_v, os_hbm.at[pl.ds(off, _CH)])
            return carry

        lax.fori_loop(0, n_chunks, body, 0)

    return gk(tgt2d, srcp2d, idx)


def _sc_gather2(table, idx):
    tot = idx.shape[0]
    per_w = tot // _NWORK
    n_chunks = per_w // _CH
    mesh = plsc.VectorSubcoreMesh(core_axis_name="c", subcore_axis_name="s")

    @functools.partial(
        pl.kernel, mesh=mesh,
        out_type=jax.ShapeDtypeStruct((tot, _C), jnp.float32),
        scratch_types=[
            pltpu.VMEM((_CH,), jnp.int32),
            pltpu.VMEM((_CH, _C), jnp.float32),
            pltpu.SemaphoreType.DMA,
        ],
    )
    def gk(table_hbm, idx_hbm, out_hbm, idx_v, rows_v, sem):
        wid = lax.axis_index("s") * 2 + lax.axis_index("c")
        base = wid * per_w

        def body(i, carry):
            off = base + i * _CH
            pltpu.sync_copy(idx_hbm.at[pl.ds(off, _CH)], idx_v)
            pltpu.async_copy(table_hbm.at[idx_v], rows_v, sem).wait()
            pltpu.sync_copy(rows_v, out_hbm.at[pl.ds(off, _CH)])
            return carry

        lax.fori_loop(0, n_chunks, body, 0)

    return gk(table, idx)


def _bf(x):
    return x.astype(jnp.bfloat16).astype(jnp.float32)


def _attn_core(t2, q, wk, wv):
    km = _dotT(t2, wk).reshape(_RB, _K, _C)
    v = _dotT(t2, wv).reshape(_RB, _K, _C)
    logits = jnp.sum(_bf(km) * _bf(q)[:, None, :], axis=-1)
    logits = logits / float(np.sqrt(_C))
    mx = jnp.max(logits, axis=-1, keepdims=True)
    e = jnp.exp(logits - mx)
    a = e / jnp.sum(e, axis=-1, keepdims=True)
    return jnp.sum(_bf(v) * _bf(a)[:, :, None], axis=1)


def _attn1_kernel(gtgt_ref, gsrc_ref, srcb_ref, tgtb_ref,
                  wq0_ref, wk0_ref, wv0_ref, wp0_ref, bp0_ref, out_ref):
    gsrc = gsrc_ref[...][:, :, :3]
    resid = (srcb_ref[0][:, None, :] - gsrc).reshape(_RB * _K, 3)
    rp = _dotT(resid, wp0_ref[...]) + bp0_ref[...]
    t2 = rp + gtgt_ref[...].reshape(_RB * _K, _C)
    q = _dotT(tgtb_ref[0], wq0_ref[...])
    out_ref[0] = _attn_core(t2, q, wk0_ref[...], wv0_ref[...])


def _attn1(gtgt3, gsrc3, src, tgt, Wq0, Wk0, Wv0, Wp0, bp0):
    wspec = pl.BlockSpec((_C, _C), lambda b, i: (0, 0))
    return pl.pallas_call(
        _attn1_kernel,
        grid=(_B, _NBLK),
        in_specs=[
            pl.BlockSpec((_RB, _K, _C), lambda b, i: (b * _NBLK + i, 0, 0)),
            pl.BlockSpec((_RB, _K, _SP), lambda b, i: (b * _NBLK + i, 0, 0)),
            pl.BlockSpec((1, _RB, 3), lambda b, i: (b, i, 0)),
            pl.BlockSpec((1, _RB, _C), lambda b, i: (b, i, 0)),
            wspec, wspec, wspec,
            pl.BlockSpec((_C, 3), lambda b, i: (0, 0)),
            pl.BlockSpec((1, _C), lambda b, i: (0, 0)),
        ],
        out_specs=pl.BlockSpec((1, _RB, _C), lambda b, i: (b, i, 0)),
        out_shape=jax.ShapeDtypeStruct((_B, _N, _C), jnp.float32),
    )(gtgt3, gsrc3, src, tgt, Wq0, Wk0, Wv0, Wp0, bp0)


def _ln(x, g, b):
    mu = jnp.mean(x, axis=-1, keepdims=True)
    xc = x - mu
    var = jnp.mean(xc * xc, axis=-1, keepdims=True)
    return xc / jnp.sqrt(var + 1e-5) * g + b


def _attn2_kernel(gout_ref, ob_ref, q2_ref, wp1_ref, bp1_ref, wk1_ref,
                  wv1_ref, tgt_ref, wl_ref, bl_ref, g0_ref, b0_ref,
                  g1_ref, b1_ref, out_ref):
    gout = gout_ref[...]
    resid = (ob_ref[0][:, None, :] - gout).reshape(_RB * _K, _C)
    rp = _dotT(resid, wp1_ref[...]) + bp1_ref[...]
    t2 = rp + gout.reshape(_RB * _K, _C)
    o2 = _attn_core(t2, q2_ref[0], wk1_ref[...], wv1_ref[...])
    tgt = tgt_ref[0]
    out0 = tgt + o2
    y0 = _ln(out0, g0_ref[...], b0_ref[...])
    out1 = _dotT(y0, wl_ref[...]) + bl_ref[...]
    out2 = tgt + out1
    out_ref[0] = _ln(out2, g1_ref[...], b1_ref[...])


def _attn2(gout3, out1, Q2, Wp1, bp1, Wk1, Wv1, tgt, Wl, bl, g0, b0, g1, b1):
    wspec = pl.BlockSpec((_C, _C), lambda b, i: (0, 0))
    vspec = pl.BlockSpec((1, _C), lambda b, i: (0, 0))
    rspec = pl.BlockSpec((1, _RB, _C), lambda b, i: (b, i, 0))
    return pl.pallas_call(
        _attn2_kernel,
        grid=(_B, _NBLK),
        in_specs=[
            pl.BlockSpec((_RB, _K, _C), lambda b, i: (b * _NBLK + i, 0, 0)),
            rspec, rspec, wspec, vspec, wspec, wspec, rspec,
            wspec, vspec, vspec, vspec, vspec, vspec,
        ],
        out_specs=rspec,
        out_shape=jax.ShapeDtypeStruct((_B, _N, _C), jnp.float32),
    )(gout3, out1, Q2, Wp1, bp1, Wk1, Wv1, tgt, Wl, bl, g0, b0, g1, b1)


def kernel(src, tgt, Wq0, Wk0, Wv0, Wq1, Wk1, Wv1, Wl, bl, Wp0, bp0, Wp1,
           bp1, g0, b0, g1, b1):
    bp0r = bp0.reshape(1, _C)
    bp1r = bp1.reshape(1, _C)
    blr = bl.reshape(1, _C)
    g0r, b0r = g0.reshape(1, _C), b0.reshape(1, _C)
    g1r, b1r = g1.reshape(1, _C), b1.reshape(1, _C)
    srcp = jnp.concatenate(
        [src, jnp.zeros((_B, _N, _SP - 3), jnp.float32)], axis=-1)

    idx1 = _knn1(src)
    gtgt, gsrc = _sc_gather1(tgt.reshape(_B * _N, _C),
                             srcp.reshape(_B * _N, _SP), idx1.reshape(-1))
    out1 = _attn1(gtgt.reshape(_B * _N, _K, _C),
                  gsrc.reshape(_B * _N, _K, _SP),
                  src, tgt, Wq0, Wk0, Wv0, Wp0, bp0r)

    idx2, Q2 = _knn2(out1, Wq1)
    gout = _sc_gather2(out1.reshape(_B * _N, _C), idx2.reshape(-1))
    y = _attn2(gout.reshape(_B * _N, _K, _C), out1, Q2, Wp1, bp1r,
               Wk1, Wv1, tgt, Wl, blr, g0r, b0r, g1r, b1r)
    return jnp.transpose(y, (1, 0, 2))


# speedup vs baseline: 1.8489x; 1.8489x over previous
import functools

import jax
import jax.numpy as jnp
import numpy as np
from jax import lax
from jax.experimental import pallas as pl
from jax.experimental.pallas import tpu as pltpu
from jax.experimental.pallas import tpu_sc as plsc

_B, _N, _C, _K = 8, 2048, 128, 16
_RB = 512
_NBLK = _N // _RB
_SP = 128


def _dotT(a, b):
    return lax.dot_general(a, b, (((1,), (1,)), ((), ())),
                           preferred_element_type=jnp.float32)


def _topk16(d2, b, iotaf, work_ref):
    work_ref[...] = d2
    cols = []
    for _ in range(_K):
        w = work_ref[...]
        m = jnp.min(w, axis=1, keepdims=True)
        jf = jnp.min(jnp.where(w <= m, iotaf, float(_N)),
                     axis=1, keepdims=True)
        cols.append(jf)
        work_ref[...] = jnp.where(iotaf == jf, jnp.inf, w)
    idx = jnp.concatenate(cols, axis=1).astype(jnp.int32)
    return idx + b * _N


def _knn1_kernel(srcb_ref, srcf_ref, idx_ref, work_ref):
    b = pl.program_id(0)
    sb = srcb_ref[0]
    sf = srcf_ref[0]
    x2b = jnp.sum(sb * sb, axis=-1, keepdims=True)
    x2f = jnp.sum(sf * sf, axis=-1, keepdims=True)
    d2 = x2b + x2f.reshape(1, _N) - 2.0 * _dotT(sb, sf)
    d2 = jnp.maximum(d2, 0.0)
    iotaf = lax.broadcasted_iota(jnp.int32, (_RB, _N), 1).astype(jnp.float32)
    idx_ref[0] = _topk16(d2, b, iotaf, work_ref)


def _knn1(src):
    bb = src.shape[0]
    return pl.pallas_call(
        _knn1_kernel,
        grid=(bb, _NBLK),
        in_specs=[
            pl.BlockSpec((1, _RB, 3), lambda b, i: (b, i, 0)),
            pl.BlockSpec((1, _N, 3), lambda b, i: (b, 0, 0)),
        ],
        out_specs=pl.BlockSpec((1, _RB, _K), lambda b, i: (b, i, 0)),
        out_shape=jax.ShapeDtypeStruct((bb, _N, _K), jnp.int32),
        scratch_shapes=[pltpu.VMEM((_RB, _N), jnp.float32)],
    )(src, src)


def _knn2_kernel(ob_ref, of_ref, wq1_ref, idx_ref, q2_ref, work_ref):
    b = pl.program_id(0)
    ob = ob_ref[0]
    of = of_ref[0]
    x2b = jnp.sum(ob * ob, axis=-1, keepdims=True)
    x2f = jnp.sum(of * of, axis=-1, keepdims=True)
    d2 = x2b + x2f.reshape(1, _N) - 2.0 * _dotT(ob, of)
    d2 = jnp.maximum(d2, 0.0)
    iotaf = lax.broadcasted_iota(jnp.int32, (_RB, _N), 1).astype(jnp.float32)
    idx_ref[0] = _topk16(d2, b, iotaf, work_ref)
    q2_ref[0] = _dotT(ob, wq1_ref[...])


def _knn2(out1, Wq1):
    bb = out1.shape[0]
    return pl.pallas_call(
        _knn2_kernel,
        grid=(bb, _NBLK),
        in_specs=[
            pl.BlockSpec((1, _RB, _C), lambda b, i: (b, i, 0)),
            pl.BlockSpec((1, _N, _C), lambda b, i: (b, 0, 0)),
            pl.BlockSpec((_C, _C), lambda b, i: (0, 0)),
        ],
        out_specs=[
            pl.BlockSpec((1, _RB, _K), lambda b, i: (b, i, 0)),
            pl.BlockSpec((1, _RB, _C), lambda b, i: (b, i, 0)),
        ],
        out_shape=[
            jax.ShapeDtypeStruct((bb, _N, _K), jnp.int32),
            jax.ShapeDtypeStruct((bb, _N, _C), jnp.float32),
        ],
        scratch_shapes=[pltpu.VMEM((_RB, _N), jnp.float32)],
    )(out1, out1, Wq1)


_NWORK = 32
_CH = 512


def _sc_gather1(tgt2d, srcp2d, idx):
    tot = idx.shape[0]
    per_w = tot // _NWORK
    n_chunks = per_w // _CH
    mesh = plsc.VectorSubcoreMesh(core_axis_name="c", subcore_axis_name="s")

    @functools.partial(
        pl.kernel, mesh=mesh,
        out_type=[
            jax.ShapeDtypeStruct((tot, _C), jnp.float32),
            jax.ShapeDtypeStruct((tot, _SP), jnp.float32),
        ],
        scratch_types=[
            pltpu.VMEM((_CH,), jnp.int32),
            pltpu.VMEM((_CH, _C), jnp.float32),
            pltpu.SemaphoreType.DMA,
        ],
    )
    def gk(tgt_hbm, srcp_hbm, idx_hbm, ot_hbm, os_hbm,
           idx_v, rows_v, sem):
        wid = lax.axis_index("s") * 2 + lax.axis_index("c")
        base = wid * per_w

        def body(i, carry):
            off = base + i * _CH
            pltpu.sync_copy(idx_hbm.at[pl.ds(off, _CH)], idx_v)
            pltpu.async_copy(tgt_hbm.at[idx_v], rows_v, sem).wait()
            pltpu.sync_copy(rows_v, ot_hbm.at[pl.ds(off, _CH)])
            pltpu.async_copy(srcp_hbm.at[idx_v], rows_v, sem).wait()
            pltpu.sync_copy(rows_v, os_hbm.at[pl.ds(off, _CH)])
            return carry

        lax.fori_loop(0, n_chunks, body, 0)

    return gk(tgt2d, srcp2d, idx)


def _sc_gather2(table, idx):
    tot = idx.shape[0]
    per_w = tot // _NWORK
    n_chunks = per_w // _CH
    mesh = plsc.VectorSubcoreMesh(core_axis_name="c", subcore_axis_name="s")

    @functools.partial(
        pl.kernel, mesh=mesh,
        out_type=jax.ShapeDtypeStruct((tot, _C), jnp.float32),
        scratch_types=[
            pltpu.VMEM((_CH,), jnp.int32),
            pltpu.VMEM((_CH, _C), jnp.float32),
            pltpu.SemaphoreType.DMA,
        ],
    )
    def gk(table_hbm, idx_hbm, out_hbm, idx_v, rows_v, sem):
        wid = lax.axis_index("s") * 2 + lax.axis_index("c")
        base = wid * per_w

        def body(i, carry):
            off = base + i * _CH
            pltpu.sync_copy(idx_hbm.at[pl.ds(off, _CH)], idx_v)
            pltpu.async_copy(table_hbm.at[idx_v], rows_v, sem).wait()
            pltpu.sync_copy(rows_v, out_hbm.at[pl.ds(off, _CH)])
            return carry

        lax.fori_loop(0, n_chunks, body, 0)

    return gk(table, idx)


def _bf(x):
    return x.astype(jnp.bfloat16).astype(jnp.float32)


def _attn_core(t2, q, wk, wv):
    km = _dotT(t2, wk).reshape(_RB, _K, _C)
    v = _dotT(t2, wv).reshape(_RB, _K, _C)
    logits = jnp.sum(_bf(km) * _bf(q)[:, None, :], axis=-1)
    logits = logits / float(np.sqrt(_C))
    mx = jnp.max(logits, axis=-1, keepdims=True)
    e = jnp.exp(logits - mx)
    a = e / jnp.sum(e, axis=-1, keepdims=True)
    return jnp.sum(_bf(v) * _bf(a)[:, :, None], axis=1)


def _attn1_kernel(gtgt_ref, gsrc_ref, srcb_ref, tgtb_ref,
                  wq0_ref, wk0_ref, wv0_ref, wp0_ref, bp0_ref, out_ref):
    gsrc = gsrc_ref[...][:, :, :3]
    resid = (srcb_ref[0][:, None, :] - gsrc).reshape(_RB * _K, 3)
    rp = _dotT(resid, wp0_ref[...]) + bp0_ref[...]
    t2 = rp + gtgt_ref[...].reshape(_RB * _K, _C)
    q = _dotT(tgtb_ref[0], wq0_ref[...])
    out_ref[0] = _attn_core(t2, q, wk0_ref[...], wv0_ref[...])


def _attn1(gtgt3, gsrc3, src, tgt, Wq0, Wk0, Wv0, Wp0, bp0):
    bb = src.shape[0]
    wspec = pl.BlockSpec((_C, _C), lambda b, i: (0, 0))
    return pl.pallas_call(
        _attn1_kernel,
        grid=(bb, _NBLK),
        in_specs=[
            pl.BlockSpec((_RB, _K, _C), lambda b, i: (b * _NBLK + i, 0, 0)),
            pl.BlockSpec((_RB, _K, _SP), lambda b, i: (b * _NBLK + i, 0, 0)),
            pl.BlockSpec((1, _RB, 3), lambda b, i: (b, i, 0)),
            pl.BlockSpec((1, _RB, _C), lambda b, i: (b, i, 0)),
            wspec, wspec, wspec,
            pl.BlockSpec((_C, 3), lambda b, i: (0, 0)),
            pl.BlockSpec((1, _C), lambda b, i: (0, 0)),
        ],
        out_specs=pl.BlockSpec((1, _RB, _C), lambda b, i: (b, i, 0)),
        out_shape=jax.ShapeDtypeStruct((bb, _N, _C), jnp.float32),
    )(gtgt3, gsrc3, src, tgt, Wq0, Wk0, Wv0, Wp0, bp0)


def _ln(x, g, b):
    mu = jnp.mean(x, axis=-1, keepdims=True)
    xc = x - mu
    var = jnp.mean(xc * xc, axis=-1, keepdims=True)
    return xc / jnp.sqrt(var + 1e-5) * g + b


def _attn2_kernel(gout_ref, ob_ref, q2_ref, wp1_ref, bp1_ref, wk1_ref,
                  wv1_ref, tgt_ref, wl_ref, bl_ref, g0_ref, b0_ref,
                  g1_ref, b1_ref, out_ref):
    gout = gout_ref[...]
    resid = (ob_ref[0][:, None, :] - gout).reshape(_RB * _K, _C)
    rp = _dotT(resid, wp1_ref[...]) + bp1_ref[...]
    t2 = rp + gout.reshape(_RB * _K, _C)
    o2 = _attn_core(t2, q2_ref[0], wk1_ref[...], wv1_ref[...])
    tgt = tgt_ref[0]
    out0 = tgt + o2
    y0 = _ln(out0, g0_ref[...], b0_ref[...])
    out1 = _dotT(y0, wl_ref[...]) + bl_ref[...]
    out2 = tgt + out1
    out_ref[0] = _ln(out2, g1_ref[...], b1_ref[...])


def _attn2(gout3, out1, Q2, Wp1, bp1, Wk1, Wv1, tgt, Wl, bl, g0, b0, g1, b1):
    bb = out1.shape[0]
    wspec = pl.BlockSpec((_C, _C), lambda b, i: (0, 0))
    vspec = pl.BlockSpec((1, _C), lambda b, i: (0, 0))
    rspec = pl.BlockSpec((1, _RB, _C), lambda b, i: (b, i, 0))
    return pl.pallas_call(
        _attn2_kernel,
        grid=(bb, _NBLK),
        in_specs=[
            pl.BlockSpec((_RB, _K, _C), lambda b, i: (b * _NBLK + i, 0, 0)),
            rspec, rspec, wspec, vspec, wspec, wspec, rspec,
            wspec, vspec, vspec, vspec, vspec, vspec,
        ],
        out_specs=rspec,
        out_shape=jax.ShapeDtypeStruct((bb, _N, _C), jnp.float32),
    )(gout3, out1, Q2, Wp1, bp1, Wk1, Wv1, tgt, Wl, bl, g0, b0, g1, b1)


def kernel(src, tgt, Wq0, Wk0, Wv0, Wq1, Wk1, Wv1, Wl, bl, Wp0, bp0, Wp1,
           bp1, g0, b0, g1, b1):
    bp0r = bp0.reshape(1, _C)
    bp1r = bp1.reshape(1, _C)
    blr = bl.reshape(1, _C)
    g0r, b0r = g0.reshape(1, _C), b0.reshape(1, _C)
    g1r, b1r = g1.reshape(1, _C), b1.reshape(1, _C)
    srcp = jnp.concatenate(
        [src, jnp.zeros((_B, _N, _SP - 3), jnp.float32)], axis=-1)

    hb = _B // 2
    halves = []
    idx1s, g1s, out1s, knn2s, gouts = {}, {}, {}, {}, {}
    for h in range(2):
        sl = slice(h * hb, (h + 1) * hb)
        idx1s[h] = _knn1(src[sl])
    for h in range(2):
        sl = slice(h * hb, (h + 1) * hb)
        g1s[h] = _sc_gather1(tgt[sl].reshape(hb * _N, _C),
                             srcp[sl].reshape(hb * _N, _SP),
                             idx1s[h].reshape(-1))
    for h in range(2):
        sl = slice(h * hb, (h + 1) * hb)
        gtgt, gsrc = g1s[h]
        out1s[h] = _attn1(gtgt.reshape(hb * _N, _K, _C),
                          gsrc.reshape(hb * _N, _K, _SP),
                          src[sl], tgt[sl], Wq0, Wk0, Wv0, Wp0, bp0r)
    for h in range(2):
        knn2s[h] = _knn2(out1s[h], Wq1)
    for h in range(2):
        idx2, _ = knn2s[h]
        gouts[h] = _sc_gather2(out1s[h].reshape(hb * _N, _C),
                               idx2.reshape(-1))
    for h in range(2):
        sl = slice(h * hb, (h + 1) * hb)
        _, Q2 = knn2s[h]
        halves.append(_attn2(gouts[h].reshape(hb * _N, _K, _C), out1s[h],
                             Q2, Wp1, bp1r, Wk1, Wv1, tgt[sl], Wl, blr,
                             g0r, b0r, g1r, b1r))
    y = jnp.concatenate(halves, axis=0)
    return jnp.transpose(y, (1, 0, 2))


# speedup vs baseline: 1.8661x; 1.0093x over previous
import functools

import jax
import jax.numpy as jnp
import numpy as np
from jax import lax
from jax.experimental import pallas as pl
from jax.experimental.pallas import tpu as pltpu
from jax.experimental.pallas import tpu_sc as plsc

_B, _N, _C, _K = 8, 2048, 128, 16
_RB = 512
_RA = 1024
_NBLK = _N // _RB
_NA = _N // _RA
_SP = 128


def _dotT(a, b):
    return lax.dot_general(a, b, (((1,), (1,)), ((), ())),
                           preferred_element_type=jnp.float32)


def _topk16(d2, b, iotaf, work_ref):
    work_ref[...] = d2
    cols = []
    for _ in range(_K):
        w = work_ref[...]
        m = jnp.min(w, axis=1, keepdims=True)
        jf = jnp.min(jnp.where(w <= m, iotaf, float(_N)),
                     axis=1, keepdims=True)
        cols.append(jf)
        work_ref[...] = jnp.where(iotaf == jf, jnp.inf, w)
    idx = jnp.concatenate(cols, axis=1).astype(jnp.int32)
    return idx + b * _N


def _knn1_kernel(srcb_ref, srcf_ref, idx_ref, work_ref):
    b = pl.program_id(0)
    sb = srcb_ref[0]
    sf = srcf_ref[0]
    x2b = jnp.sum(sb * sb, axis=-1, keepdims=True)
    x2f = jnp.sum(sf * sf, axis=-1, keepdims=True)
    d2 = x2b + x2f.reshape(1, _N) - 2.0 * _dotT(sb, sf)
    d2 = jnp.maximum(d2, 0.0)
    iotaf = lax.broadcasted_iota(jnp.int32, (_RB, _N), 1).astype(jnp.float32)
    idx_ref[0] = _topk16(d2, b, iotaf, work_ref)


def _knn1(src):
    bb = src.shape[0]
    return pl.pallas_call(
        _knn1_kernel,
        grid=(bb, _NBLK),
        in_specs=[
            pl.BlockSpec((1, _RB, 3), lambda b, i: (b, i, 0)),
            pl.BlockSpec((1, _N, 3), lambda b, i: (b, 0, 0)),
        ],
        out_specs=pl.BlockSpec((1, _RB, _K), lambda b, i: (b, i, 0)),
        out_shape=jax.ShapeDtypeStruct((bb, _N, _K), jnp.int32),
        scratch_shapes=[pltpu.VMEM((_RB, _N), jnp.float32)],
    )(src, src)


def _knn2_kernel(ob_ref, of_ref, wq1_ref, idx_ref, q2_ref, work_ref):
    b = pl.program_id(0)
    ob = ob_ref[0]
    of = of_ref[0]
    x2b = jnp.sum(ob * ob, axis=-1, keepdims=True)
    x2f = jnp.sum(of * of, axis=-1, keepdims=True)
    d2 = x2b + x2f.reshape(1, _N) - 2.0 * _dotT(ob, of)
    d2 = jnp.maximum(d2, 0.0)
    iotaf = lax.broadcasted_iota(jnp.int32, (_RB, _N), 1).astype(jnp.float32)
    idx_ref[0] = _topk16(d2, b, iotaf, work_ref)
    q2_ref[0] = _dotT(ob, wq1_ref[...])


def _knn2(out1, Wq1):
    bb = out1.shape[0]
    return pl.pallas_call(
        _knn2_kernel,
        grid=(bb, _NBLK),
        in_specs=[
            pl.BlockSpec((1, _RB, _C), lambda b, i: (b, i, 0)),
            pl.BlockSpec((1, _N, _C), lambda b, i: (b, 0, 0)),
            pl.BlockSpec((_C, _C), lambda b, i: (0, 0)),
        ],
        out_specs=[
            pl.BlockSpec((1, _RB, _K), lambda b, i: (b, i, 0)),
            pl.BlockSpec((1, _RB, _C), lambda b, i: (b, i, 0)),
        ],
        out_shape=[
            jax.ShapeDtypeStruct((bb, _N, _K), jnp.int32),
            jax.ShapeDtypeStruct((bb, _N, _C), jnp.float32),
        ],
        scratch_shapes=[pltpu.VMEM((_RB, _N), jnp.float32)],
    )(out1, out1, Wq1)


_NWORK = 32
_CH = 512


def _sc_gather1(tgt2d, srcp2d, idx):
    tot = idx.shape[0]
    per_w = tot // _NWORK
    n_chunks = per_w // _CH
    mesh = plsc.VectorSubcoreMesh(core_axis_name="c", subcore_axis_name="s")

    @functools.partial(
        pl.kernel, mesh=mesh,
        out_type=[
            jax.ShapeDtypeStruct((tot, _C), jnp.float32),
            jax.ShapeDtypeStruct((tot, _SP), jnp.float32),
        ],
        scratch_types=[
            pltpu.VMEM((_CH,), jnp.int32),
            pltpu.VMEM((_CH, _C), jnp.float32),
            pltpu.SemaphoreType.DMA,
        ],
    )
    def gk(tgt_hbm, srcp_hbm, idx_hbm, ot_hbm, os_hbm,
           idx_v, rows_v, sem):
        wid = lax.axis_index("s") * 2 + lax.axis_index("c")
        base = wid * per_w

        def body(i, carry):
            off = base + i * _CH
            pltpu.sync_copy(idx_hbm.at[pl.ds(off, _CH)], idx_v)
            pltpu.async_copy(tgt_hbm.at[idx_v], rows_v, sem).wait()
            pltpu.sync_copy(rows_v, ot_hbm.at[pl.ds(off, _CH)])
            pltpu.async_copy(srcp_hbm.at[idx_v], rows_v, sem).wait()
            pltpu.sync_copy(rows_v, os_hbm.at[pl.ds(off, _CH)])
            return carry

        lax.fori_loop(0, n_chunks, body, 0)

    return gk(tgt2d, srcp2d, idx)


def _sc_gather2(table, idx):
    tot = idx.shape[0]
    per_w = tot // _NWORK
    n_chunks = per_w // _CH
    mesh = plsc.VectorSubcoreMesh(core_axis_name="c", subcore_axis_name="s")

    @functools.partial(
        pl.kernel, mesh=mesh,
        out_type=jax.ShapeDtypeStruct((tot, _C), jnp.float32),
        scratch_types=[
            pltpu.VMEM((_CH,), jnp.int32),
            pltpu.VMEM((_CH, _C), jnp.float32),
            pltpu.SemaphoreType.DMA,
        ],
    )
    def gk(table_hbm, idx_hbm, out_hbm, idx_v, rows_v, sem):
        wid = lax.axis_index("s") * 2 + lax.axis_index("c")
        base = wid * per_w

        def body(i, carry):
            off = base + i * _CH
            pltpu.sync_copy(idx_hbm.at[pl.ds(off, _CH)], idx_v)
            pltpu.async_copy(table_hbm.at[idx_v], rows_v, sem).wait()
            pltpu.sync_copy(rows_v, out_hbm.at[pl.ds(off, _CH)])
            return carry

        lax.fori_loop(0, n_chunks, body, 0)

    return gk(table, idx)


def _bf(x):
    return x.astype(jnp.bfloat16).astype(jnp.float32)


def _attn_core(t2, q, wk, wv):
    km = _dotT(t2, wk).reshape(_RA, _K, _C)
    v = _dotT(t2, wv).reshape(_RA, _K, _C)
    logits = jnp.sum(_bf(km) * _bf(q)[:, None, :], axis=-1)
    logits = logits / float(np.sqrt(_C))
    mx = jnp.max(logits, axis=-1, keepdims=True)
    e = jnp.exp(logits - mx)
    a = e / jnp.sum(e, axis=-1, keepdims=True)
    return jnp.sum(_bf(v) * _bf(a)[:, :, None], axis=1)


def _attn1_kernel(gtgt_ref, gsrc_ref, srcb_ref, tgtb_ref,
                  wq0_ref, wk0_ref, wv0_ref, wp0_ref, bp0_ref, out_ref):
    gsrc = gsrc_ref[...][:, :, :3]
    resid = (srcb_ref[0][:, None, :] - gsrc).reshape(_RA * _K, 3)
    rp = _dotT(resid, wp0_ref[...]) + bp0_ref[...]
    t2 = rp + gtgt_ref[...].reshape(_RA * _K, _C)
    q = _dotT(tgtb_ref[0], wq0_ref[...])
    out_ref[0] = _attn_core(t2, q, wk0_ref[...], wv0_ref[...])


def _attn1(gtgt3, gsrc3, src, tgt, Wq0, Wk0, Wv0, Wp0, bp0):
    bb = src.shape[0]
    wspec = pl.BlockSpec((_C, _C), lambda b, i: (0, 0))
    return pl.pallas_call(
        _attn1_kernel,
        grid=(bb, _NA),
        in_specs=[
            pl.BlockSpec((_RA, _K, _C), lambda b, i: (b * _NA + i, 0, 0)),
            pl.BlockSpec((_RA, _K, _SP), lambda b, i: (b * _NA + i, 0, 0)),
            pl.BlockSpec((1, _RA, 3), lambda b, i: (b, i, 0)),
            pl.BlockSpec((1, _RA, _C), lambda b, i: (b, i, 0)),
            wspec, wspec, wspec,
            pl.BlockSpec((_C, 3), lambda b, i: (0, 0)),
            pl.BlockSpec((1, _C), lambda b, i: (0, 0)),
        ],
        out_specs=pl.BlockSpec((1, _RA, _C), lambda b, i: (b, i, 0)),
        out_shape=jax.ShapeDtypeStruct((bb, _N, _C), jnp.float32),
    )(gtgt3, gsrc3, src, tgt, Wq0, Wk0, Wv0, Wp0, bp0)


def _ln(x, g, b):
    mu = jnp.mean(x, axis=-1, keepdims=True)
    xc = x - mu
    var = jnp.mean(xc * xc, axis=-1, keepdims=True)
    return xc / jnp.sqrt(var + 1e-5) * g + b


def _attn2_kernel(gout_ref, ob_ref, q2_ref, wp1_ref, bp1_ref, wk1_ref,
                  wv1_ref, tgt_ref, wl_ref, bl_ref, g0_ref, b0_ref,
                  g1_ref, b1_ref, out_ref):
    gout = gout_ref[...]
    resid = (ob_ref[0][:, None, :] - gout).reshape(_RA * _K, _C)
    rp = _dotT(resid, wp1_ref[...]) + bp1_ref[...]
    t2 = rp + gout.reshape(_RA * _K, _C)
    o2 = _attn_core(t2, q2_ref[0], wk1_ref[...], wv1_ref[...])
    tgt = tgt_ref[0]
    out0 = tgt + o2
    y0 = _ln(out0, g0_ref[...], b0_ref[...])
    out1 = _dotT(y0, wl_ref[...]) + bl_ref[...]
    out2 = tgt + out1
    out_ref[0] = _ln(out2, g1_ref[...], b1_ref[...])


def _attn2(gout3, out1, Q2, Wp1, bp1, Wk1, Wv1, tgt, Wl, bl, g0, b0, g1, b1):
    bb = out1.shape[0]
    wspec = pl.BlockSpec((_C, _C), lambda b, i: (0, 0))
    vspec = pl.BlockSpec((1, _C), lambda b, i: (0, 0))
    rspec = pl.BlockSpec((1, _RA, _C), lambda b, i: (b, i, 0))
    return pl.pallas_call(
        _attn2_kernel,
        grid=(bb, _NA),
        in_specs=[
            pl.BlockSpec((_RA, _K, _C), lambda b, i: (b * _NA + i, 0, 0)),
            rspec, rspec, wspec, vspec, wspec, wspec, rspec,
            wspec, vspec, vspec, vspec, vspec, vspec,
        ],
        out_specs=rspec,
        out_shape=jax.ShapeDtypeStruct((bb, _N, _C), jnp.float32),
    )(gout3, out1, Q2, Wp1, bp1, Wk1, Wv1, tgt, Wl, bl, g0, b0, g1, b1)


def kernel(src, tgt, Wq0, Wk0, Wv0, Wq1, Wk1, Wv1, Wl, bl, Wp0, bp0, Wp1,
           bp1, g0, b0, g1, b1):
    bp0r = bp0.reshape(1, _C)
    bp1r = bp1.reshape(1, _C)
    blr = bl.reshape(1, _C)
    g0r, b0r = g0.reshape(1, _C), b0.reshape(1, _C)
    g1r, b1r = g1.reshape(1, _C), b1.reshape(1, _C)
    srcp = jnp.concatenate(
        [src, jnp.zeros((_B, _N, _SP - 3), jnp.float32)], axis=-1)

    hb = _B // 2
    halves = []
    idx1s, g1s, out1s, knn2s, gouts = {}, {}, {}, {}, {}
    for h in range(2):
        sl = slice(h * hb, (h + 1) * hb)
        idx1s[h] = _knn1(src[sl])
    for h in range(2):
        sl = slice(h * hb, (h + 1) * hb)
        g1s[h] = _sc_gather1(tgt[sl].reshape(hb * _N, _C),
                             srcp[sl].reshape(hb * _N, _SP),
                             idx1s[h].reshape(-1))
    for h in range(2):
        sl = slice(h * hb, (h + 1) * hb)
        gtgt, gsrc = g1s[h]
        out1s[h] = _attn1(gtgt.reshape(hb * _N, _K, _C),
                          gsrc.reshape(hb * _N, _K, _SP),
                          src[sl], tgt[sl], Wq0, Wk0, Wv0, Wp0, bp0r)
    for h in range(2):
        knn2s[h] = _knn2(out1s[h], Wq1)
    for h in range(2):
        idx2, _ = knn2s[h]
        gouts[h] = _sc_gather2(out1s[h].reshape(hb * _N, _C),
                               idx2.reshape(-1))
    for h in range(2):
        sl = slice(h * hb, (h + 1) * hb)
        _, Q2 = knn2s[h]
        halves.append(_attn2(gouts[h].reshape(hb * _N, _K, _C), out1s[h],
                             Q2, Wp1, bp1r, Wk1, Wv1, tgt[sl], Wl, blr,
                             g0r, b0r, g1r, b1r))
    y = jnp.concatenate(halves, axis=0)
    return jnp.transpose(y, (1, 0, 2))
